# baseline (speedup 1.0000x reference)
import functools

import jax
import jax.numpy as jnp
from jax import lax
from jax.experimental import pallas as pl
from jax.experimental.pallas import tpu as pltpu
from jax.experimental.pallas import tpu_sc as plsc

DIM = 64
NBUF = 8


@functools.partial(jax.jit, static_argnames=("n_chunks", "chunk"))
def _sc_gather(table, idx3, n_chunks, chunk):
    nw = idx3.shape[0]
    b_per_w = n_chunks * chunk
    b_total = nw * b_per_w
    n_rounds = n_chunks // NBUF
    mesh = plsc.VectorSubcoreMesh(core_axis_name="c", subcore_axis_name="s")

    @functools.partial(
        pl.kernel,
        mesh=mesh,
        out_type=jax.ShapeDtypeStruct((b_total, DIM), jnp.float32),
        scratch_types=[
            pltpu.VMEM((n_chunks, chunk), jnp.int32),
            pltpu.VMEM((NBUF, chunk, DIM), jnp.float32),
        ]
        + [pltpu.SemaphoreType.DMA] * (2 * NBUF),
        compiler_params=pltpu.CompilerParams(use_tc_tiling_on_sc=False),
    )
    def k(table_hbm, idx_hbm, out_hbm, idx_v, rows_v, *sems):
        gs, ws = sems[:NBUF], sems[NBUF:]
        nc = lax.axis_size("c")
        wid = lax.axis_index("s") * nc + lax.axis_index("c")
        base = wid * b_per_w
        pltpu.sync_copy(idx_hbm.at[wid], idx_v)

        def start_gather(b, j):
            pltpu.async_copy(table_hbm.at[idx_v.at[j]], rows_v.at[b], gs[b])

        def wait_gather(b):
            pltpu.make_async_copy(
                table_hbm.at[idx_v.at[0]], rows_v.at[b], gs[b]
            ).wait()

        def start_write(b, j):
            pltpu.async_copy(
                rows_v.at[b], out_hbm.at[pl.ds(base + j * chunk, chunk)], ws[b]
            )

        def wait_write(b):
            pltpu.make_async_copy(
                rows_v.at[b], out_hbm.at[pl.ds(base, chunk)], ws[b]
            ).wait()

        for b in range(NBUF - 1):
            start_gather(b, b)

        def round_body(r, carry):
            for b in range(NBUF):
                j = r * NBUF + b
                wait_gather(b)
                start_write(b, j)
                b2 = (b + NBUF - 1) % NBUF
                if b == 0:
                    @pl.when(r >= 1)
                    def _():
                        wait_write(b2)

                    start_gather(b2, j + NBUF - 1)
                else:
                    wait_write(b2)

                    @pl.when(j + NBUF - 1 < n_chunks)
                    def _():
                        start_gather(b2, j + NBUF - 1)
            return carry

        lax.fori_loop(0, n_rounds, round_body, 0)
        wait_write((n_chunks - 1) % NBUF)

    return k(table, idx3)


def kernel(source, table):
    seq, batch, one = source.shape
    b_total = seq * batch * one
    nw = 32
    chunk = 128
    n_chunks = b_total // (nw * chunk)
    idx3 = source.reshape(nw, n_chunks, chunk)
    out = _sc_gather(table, idx3, n_chunks, chunk)
    return out.reshape(seq, batch, one, DIM)
